# baseline (speedup 1.0000x reference)
import jax
import jax.numpy as jnp
from jax import lax
from jax.experimental import pallas as pl
from jax.experimental.pallas import tpu as pltpu
from jax.experimental.pallas import tpu_sc as plsc

N = 10000
E = 320000
D = 128
NC = 2
NS = 16
NW = NC * NS
LPR = 128
RW = 81
ERP = NW * RW
EP = ERP * LPR
NP = 10240
NT = NP // NS
RB = 632
RBL = N - RB * (NS - 1)

_mesh = plsc.VectorSubcoreMesh(
    core_axis_name="c", subcore_axis_name="s", num_cores=NC, num_subcores=NS
)


def _leaky(x):
    return jnp.where(x >= 0, x, 0.01 * x)


def _sc_degrees(src3d, dst3d, zn, cnt_out, is3, id3, ones_v, csrc, cdst, csem):
    c = lax.axis_index("c")
    s = lax.axis_index("s")
    wid = s * NC + c

    def fill(j, carry):
        ones_v[pl.ds(j * 16, 16)] = jnp.ones((16,), jnp.float32)
        return carry

    lax.fori_loop(0, LPR // 16, fill, 0)
    sl = pl.ds(s * NT, NT)
    pltpu.sync_copy(zn.at[sl], csrc.at[sl])
    pltpu.sync_copy(zn.at[sl], cdst.at[sl])
    plsc.subcore_barrier()

    base = wid * RW
    pltpu.sync_copy(src3d.at[pl.ds(base, RW)], is3)
    pltpu.sync_copy(dst3d.at[pl.ds(base, RW)], id3)

    def chunk(k, carry):
        i0 = 4 * k
        for j in range(4):
            pltpu.async_copy(ones_v, csrc.at[is3.at[i0 + j, 0]], csem,
                             add=True)
            pltpu.async_copy(ones_v, cdst.at[id3.at[i0 + j, 0]], csem,
                             add=True)
        for j in range(4):
            pltpu.make_async_copy(ones_v, csrc.at[is3.at[i0 + j, 0]],
                                  csem).wait()
            pltpu.make_async_copy(ones_v, cdst.at[id3.at[i0 + j, 0]],
                                  csem).wait()
        return carry

    lax.fori_loop(0, RW // 4, chunk, 0)
    for i in range(4 * (RW // 4), RW):
        pltpu.sync_copy(ones_v, csrc.at[is3.at[i, 0]], add=True)
        pltpu.sync_copy(ones_v, cdst.at[id3.at[i, 0]], add=True)
    plsc.subcore_barrier()
    pltpu.sync_copy(csrc.at[sl], cnt_out.at[0, c, sl])
    pltpu.sync_copy(cdst.at[sl], cnt_out.at[1, c, sl])


_deg_call = pl.kernel(
    _sc_degrees,
    out_type=jax.ShapeDtypeStruct((2, NC, NP), jnp.float32),
    mesh=_mesh,
    scratch_types=[
        pltpu.VMEM((RW, 1, LPR), jnp.int32),
        pltpu.VMEM((RW, 1, LPR), jnp.int32),
        pltpu.VMEM((LPR,), jnp.float32),
        pltpu.VMEM_SHARED((NP,), jnp.float32),
        pltpu.VMEM_SHARED((NP,), jnp.float32),
        pltpu.SemaphoreType.DMA,
    ],
)


def _sc_agg(z_hbm, src3d, dst3d, agg_out, ss3, dd3, b0, b1, b2,
            shared, g0, g1, g2, i0s, i1s, i2s, i3s, i4s, i5s, zs):
    c = lax.axis_index("c")
    s = lax.axis_index("s")
    wid = s * NC + c
    base = wid * RW

    ZPAD = NP - N

    def zero_rows(dst_base, nrows):
        done = 0
        while done < nrows:
            step = min(ZPAD, nrows - done)
            pltpu.async_copy(z_hbm.at[pl.ds(N, step)],
                             shared.at[pl.ds(dst_base + done, step)], zs)
            done += step
        return (nrows + ZPAD - 1) // ZPAD

    @pl.when(s < NS - 1)
    def _():
        zero_rows(s * RB, RB)

    @pl.when(s == NS - 1)
    def _():
        zero_rows((NS - 1) * RB, RBL)

    bufs = (b0, b1, b2)
    gsems = (g0, g1, g2)
    isems = (i0s, i1s, i2s, i3s, i4s, i5s)

    def i_start(i, slot):
        pltpu.async_copy(src3d.at[base + i, 0], ss3.at[slot, 0], isems[slot])
        pltpu.async_copy(dst3d.at[base + i, 0], dd3.at[slot, 0], isems[slot])

    def i_wait(i, slot):
        pltpu.make_async_copy(src3d.at[base + i, 0], ss3.at[slot, 0],
                              isems[slot]).wait()
        pltpu.make_async_copy(dst3d.at[base + i, 0], dd3.at[slot, 0],
                              isems[slot]).wait()

    def g_start(slot, bj):
        pltpu.async_copy(z_hbm.at[ss3.at[slot, 0]], bufs[bj], gsems[bj])

    def g_wait(bj):
        pltpu.make_async_copy(z_hbm.at[ss3.at[0, 0]], bufs[bj],
                              gsems[bj]).wait()

    for j in range(6):
        i_start(j, j)
    for j in range(3):
        i_wait(j, j)
        g_start(j, j)

    def zero_wait(dst_base, nrows):
        done = 0
        while done < nrows:
            step = min(ZPAD, nrows - done)
            pltpu.make_async_copy(
                z_hbm.at[pl.ds(N, step)],
                shared.at[pl.ds(dst_base + done, step)], zs).wait()
            done += step

    @pl.when(s < NS - 1)
    def _():
        zero_wait(s * RB, RB)

    @pl.when(s == NS - 1)
    def _():
        zero_wait((NS - 1) * RB, RBL)

    plsc.subcore_barrier()

    def chunk(k, carry):
        ibase = 6 * k
        for u in range(6):
            i = ibase + u
            bj = u % 3
            g_wait(bj)
            pltpu.sync_copy(bufs[bj], shared.at[dd3.at[u, 0]], add=True)

            @pl.when(i + 3 < RW)
            def _(i=i, u=u, bj=bj):
                i_wait(i + 3, (u + 3) % 6)
                g_start((u + 3) % 6, bj)

            @pl.when(i + 6 < RW)
            def _(i=i, u=u):
                i_start(i + 6, u)

        return carry

    lax.fori_loop(0, RW // 6, chunk, 0)
    for u in range(RW - 6 * (RW // 6)):
        g_wait(u % 3)
        pltpu.sync_copy(bufs[u % 3], shared.at[dd3.at[u, 0]], add=True)

    plsc.subcore_barrier()

    @pl.when(s < NS - 1)
    def _():
        sl = pl.ds(s * RB, RB)
        pltpu.sync_copy(shared.at[sl], agg_out.at[c, sl])

    @pl.when(s == NS - 1)
    def _():
        sl = pl.ds((NS - 1) * RB, RBL)
        pltpu.sync_copy(shared.at[sl], agg_out.at[c, sl])


_agg_call = pl.kernel(
    _sc_agg,
    out_type=jax.ShapeDtypeStruct((NC, N, D), jnp.float32),
    mesh=_mesh,
    scratch_types=[
        pltpu.VMEM((6, 1, LPR), jnp.int32),
        pltpu.VMEM((6, 1, LPR), jnp.int32),
        pltpu.VMEM((LPR, D), jnp.float32),
        pltpu.VMEM((LPR, D), jnp.float32),
        pltpu.VMEM((LPR, D), jnp.float32),
        pltpu.VMEM_SHARED((N, D), jnp.float32),
    ] + [pltpu.SemaphoreType.DMA] * 10,
)


def _tc_first_body(x_ref, w_ref, cnt_ref, z_ref, ns_ref, nd_ref):
    cnt = cnt_ref[...]
    deg_o = cnt[0, 0, :N] + cnt[0, 1, :N]
    row = lax.broadcasted_iota(jnp.int32, (N,), 0)
    corr = jnp.where(row < (EP - E) % N, 2.0, 1.0)
    deg_i = cnt[1, 0, :N] + cnt[1, 1, :N] - corr
    ns = lax.rsqrt(jnp.maximum(deg_o, 1.0))[:, None]
    nd = lax.rsqrt(jnp.maximum(deg_i, 1.0))[:, None]
    u = jnp.dot(x_ref[...], w_ref[...], preferred_element_type=jnp.float32)
    z_ref[:N, :] = u * ns
    z_ref[N:, :] = jnp.zeros((NP - N, D), jnp.float32)
    ns_ref[...] = ns
    nd_ref[...] = nd


_tc_first = pl.pallas_call(
    _tc_first_body,
    out_shape=(
        jax.ShapeDtypeStruct((NP, D), jnp.float32),
        jax.ShapeDtypeStruct((N, 1), jnp.float32),
        jax.ShapeDtypeStruct((N, 1), jnp.float32),
    ),
)


def _tc_mid_body(p_ref, nd_ref, b_ref, w_ref, ns_ref, z_ref):
    agg = (p_ref[0] + p_ref[1]) * nd_ref[...]
    h = _leaky(agg + b_ref[...][None, :])
    z_ref[:N, :] = (
        jnp.dot(h, w_ref[...], preferred_element_type=jnp.float32)
        * ns_ref[...]
    )
    z_ref[N:, :] = jnp.zeros((NP - N, D), jnp.float32)


_tc_mid = pl.pallas_call(
    _tc_mid_body,
    out_shape=jax.ShapeDtypeStruct((NP, D), jnp.float32),
)


def _tc_final_body(p_ref, nd_ref, b_ref, o_ref):
    agg = (p_ref[0] + p_ref[1]) * nd_ref[...]
    o_ref[...] = _leaky(agg + b_ref[...][None, :])


_tc_final = pl.pallas_call(
    _tc_final_body,
    out_shape=jax.ShapeDtypeStruct((N, D), jnp.float32),
)


def kernel(n_feat, edge_index, W1, b1, W2, b2, W3, b3):
    npad = EP - E
    src_pad = N + (jnp.arange(npad, dtype=jnp.int32) % (NP - N))
    dst_pad = jnp.arange(npad, dtype=jnp.int32) % N
    src3d = jnp.concatenate([edge_index[0], src_pad]).reshape(ERP, 1, LPR)
    dst3d = jnp.concatenate([edge_index[1], dst_pad]).reshape(ERP, 1, LPR)
    zn = jnp.zeros((NP,), jnp.float32)

    cnt = _deg_call(src3d, dst3d, zn)
    z, ns, nd = _tc_first(n_feat, W1, cnt)
    p = _agg_call(z, src3d, dst3d)
    z = _tc_mid(p, nd, b1, W2, ns)
    p = _agg_call(z, src3d, dst3d)
    z = _tc_mid(p, nd, b2, W3, ns)
    p = _agg_call(z, src3d, dst3d)
    return _tc_final(p, nd, b3)


# speedup vs baseline: 1.0447x; 1.0447x over previous
import jax
import jax.numpy as jnp
from jax import lax
from jax.experimental import pallas as pl
from jax.experimental.pallas import tpu as pltpu
from jax.experimental.pallas import tpu_sc as plsc

N = 10000
E = 320000
D = 128
NC = 2
NS = 16
NW = NC * NS
LPR = 128
RW = 81
ERP = NW * RW
EP = ERP * LPR
NP = 10240
NT = NP // NS
RB = 632
RBL = N - RB * (NS - 1)

_mesh = plsc.VectorSubcoreMesh(
    core_axis_name="c", subcore_axis_name="s", num_cores=NC, num_subcores=NS
)


def _leaky(x):
    return jnp.where(x >= 0, x, 0.01 * x)


def _sc_degrees(src3d, dst3d, zn, cnt_out, is3, id3, ones_v, csrc, cdst, csem):
    c = lax.axis_index("c")
    s = lax.axis_index("s")
    wid = s * NC + c

    def fill(j, carry):
        ones_v[pl.ds(j * 16, 16)] = jnp.ones((16,), jnp.float32)
        return carry

    lax.fori_loop(0, LPR // 16, fill, 0)
    sl = pl.ds(s * NT, NT)
    pltpu.sync_copy(zn.at[sl], csrc.at[sl])
    pltpu.sync_copy(zn.at[sl], cdst.at[sl])
    plsc.subcore_barrier()

    base = wid * RW
    pltpu.sync_copy(src3d.at[pl.ds(base, RW)], is3)
    pltpu.sync_copy(dst3d.at[pl.ds(base, RW)], id3)

    def chunk(k, carry):
        i0 = 4 * k
        for j in range(4):
            pltpu.async_copy(ones_v, csrc.at[is3.at[i0 + j, 0]], csem,
                             add=True)
            pltpu.async_copy(ones_v, cdst.at[id3.at[i0 + j, 0]], csem,
                             add=True)
        for j in range(4):
            pltpu.make_async_copy(ones_v, csrc.at[is3.at[i0 + j, 0]],
                                  csem).wait()
            pltpu.make_async_copy(ones_v, cdst.at[id3.at[i0 + j, 0]],
                                  csem).wait()
        return carry

    lax.fori_loop(0, RW // 4, chunk, 0)
    for i in range(4 * (RW // 4), RW):
        pltpu.sync_copy(ones_v, csrc.at[is3.at[i, 0]], add=True)
        pltpu.sync_copy(ones_v, cdst.at[id3.at[i, 0]], add=True)
    plsc.subcore_barrier()
    pltpu.sync_copy(csrc.at[sl], cnt_out.at[0, c, sl])
    pltpu.sync_copy(cdst.at[sl], cnt_out.at[1, c, sl])


_deg_call = pl.kernel(
    _sc_degrees,
    out_type=jax.ShapeDtypeStruct((2, NC, NP), jnp.float32),
    mesh=_mesh,
    scratch_types=[
        pltpu.VMEM((RW, 1, LPR), jnp.int32),
        pltpu.VMEM((RW, 1, LPR), jnp.int32),
        pltpu.VMEM((LPR,), jnp.float32),
        pltpu.VMEM_SHARED((NP,), jnp.float32),
        pltpu.VMEM_SHARED((NP,), jnp.float32),
        pltpu.SemaphoreType.DMA,
    ],
)


def _sc_agg(z_hbm, src3d, dst3d, zrows, agg_out, ss3, dd3, b0, b1, b2,
            shared, g0, g1, g2, i0s, i1s, i2s, i3s, i4s, i5s, zs):
    c = lax.axis_index("c")
    s = lax.axis_index("s")
    wid = s * NC + c
    base = wid * RW

    @pl.when(s < NS - 1)
    def _():
        sl = pl.ds(s * RB, RB)
        pltpu.async_copy(zrows.at[sl], shared.at[sl], zs)

    @pl.when(s == NS - 1)
    def _():
        sl = pl.ds((NS - 1) * RB, RBL)
        pltpu.async_copy(zrows.at[sl], shared.at[sl], zs)

    bufs = (b0, b1, b2)
    gsems = (g0, g1, g2)
    isems = (i0s, i1s, i2s, i3s, i4s, i5s)

    def i_start(i, slot):
        pltpu.async_copy(src3d.at[base + i, 0], ss3.at[slot, 0], isems[slot])
        pltpu.async_copy(dst3d.at[base + i, 0], dd3.at[slot, 0], isems[slot])

    def i_wait(i, slot):
        pltpu.make_async_copy(src3d.at[base + i, 0], ss3.at[slot, 0],
                              isems[slot]).wait()
        pltpu.make_async_copy(dst3d.at[base + i, 0], dd3.at[slot, 0],
                              isems[slot]).wait()

    def g_start(slot, bj):
        pltpu.async_copy(z_hbm.at[ss3.at[slot, 0]], bufs[bj], gsems[bj])

    def g_wait(bj):
        pltpu.make_async_copy(z_hbm.at[ss3.at[0, 0]], bufs[bj],
                              gsems[bj]).wait()

    for j in range(6):
        i_start(j, j)
    for j in range(3):
        i_wait(j, j)
        g_start(j, j)

    @pl.when(s < NS - 1)
    def _():
        sl = pl.ds(s * RB, RB)
        pltpu.make_async_copy(zrows.at[sl], shared.at[sl], zs).wait()

    @pl.when(s == NS - 1)
    def _():
        sl = pl.ds((NS - 1) * RB, RBL)
        pltpu.make_async_copy(zrows.at[sl], shared.at[sl], zs).wait()

    plsc.subcore_barrier()

    def chunk(k, carry):
        ibase = 6 * k
        for u in range(6):
            i = ibase + u
            bj = u % 3
            g_wait(bj)
            pltpu.sync_copy(bufs[bj], shared.at[dd3.at[u, 0]], add=True)

            @pl.when(i + 3 < RW)
            def _(i=i, u=u, bj=bj):
                i_wait(i + 3, (u + 3) % 6)
                g_start((u + 3) % 6, bj)

            @pl.when(i + 6 < RW)
            def _(i=i, u=u):
                i_start(i + 6, u)

        return carry

    lax.fori_loop(0, RW // 6, chunk, 0)
    for u in range(RW - 6 * (RW // 6)):
        g_wait(u % 3)
        pltpu.sync_copy(bufs[u % 3], shared.at[dd3.at[u, 0]], add=True)

    plsc.subcore_barrier()

    @pl.when(s < NS - 1)
    def _():
        sl = pl.ds(s * RB, RB)
        pltpu.sync_copy(shared.at[sl], agg_out.at[c, sl])

    @pl.when(s == NS - 1)
    def _():
        sl = pl.ds((NS - 1) * RB, RBL)
        pltpu.sync_copy(shared.at[sl], agg_out.at[c, sl])


_agg_call = pl.kernel(
    _sc_agg,
    out_type=jax.ShapeDtypeStruct((NC, N, D), jnp.float32),
    mesh=_mesh,
    scratch_types=[
        pltpu.VMEM((6, 1, LPR), jnp.int32),
        pltpu.VMEM((6, 1, LPR), jnp.int32),
        pltpu.VMEM((LPR, D), jnp.float32),
        pltpu.VMEM((LPR, D), jnp.float32),
        pltpu.VMEM((LPR, D), jnp.float32),
        pltpu.VMEM_SHARED((N, D), jnp.float32),
    ] + [pltpu.SemaphoreType.DMA] * 10,
)


def _tc_first_body(x_ref, w_ref, cnt_ref, z_ref, ns_ref, nd_ref):
    cnt = cnt_ref[...]
    deg_o = cnt[0, 0, :N] + cnt[0, 1, :N]
    row = lax.broadcasted_iota(jnp.int32, (N,), 0)
    corr = jnp.where(row < (EP - E) % N, 2.0, 1.0)
    deg_i = cnt[1, 0, :N] + cnt[1, 1, :N] - corr
    ns = lax.rsqrt(jnp.maximum(deg_o, 1.0))[:, None]
    nd = lax.rsqrt(jnp.maximum(deg_i, 1.0))[:, None]
    u = jnp.dot(x_ref[...], w_ref[...], preferred_element_type=jnp.float32)
    z_ref[:N, :] = u * ns
    z_ref[N:, :] = jnp.zeros((NP - N, D), jnp.float32)
    ns_ref[...] = ns
    nd_ref[...] = nd


_tc_first = pl.pallas_call(
    _tc_first_body,
    out_shape=(
        jax.ShapeDtypeStruct((NP, D), jnp.float32),
        jax.ShapeDtypeStruct((N, 1), jnp.float32),
        jax.ShapeDtypeStruct((N, 1), jnp.float32),
    ),
)


def _tc_mid_body(p_ref, nd_ref, b_ref, w_ref, ns_ref, z_ref):
    agg = (p_ref[0] + p_ref[1]) * nd_ref[...]
    h = _leaky(agg + b_ref[...][None, :])
    z_ref[:N, :] = (
        jnp.dot(h, w_ref[...], preferred_element_type=jnp.float32)
        * ns_ref[...]
    )
    z_ref[N:, :] = jnp.zeros((NP - N, D), jnp.float32)


_tc_mid = pl.pallas_call(
    _tc_mid_body,
    out_shape=jax.ShapeDtypeStruct((NP, D), jnp.float32),
)


def _tc_final_body(p_ref, nd_ref, b_ref, o_ref):
    agg = (p_ref[0] + p_ref[1]) * nd_ref[...]
    o_ref[...] = _leaky(agg + b_ref[...][None, :])


_tc_final = pl.pallas_call(
    _tc_final_body,
    out_shape=jax.ShapeDtypeStruct((N, D), jnp.float32),
)


def kernel(n_feat, edge_index, W1, b1, W2, b2, W3, b3):
    npad = EP - E
    src_pad = N + (jnp.arange(npad, dtype=jnp.int32) % (NP - N))
    dst_pad = jnp.arange(npad, dtype=jnp.int32) % N
    src3d = jnp.concatenate([edge_index[0], src_pad]).reshape(ERP, 1, LPR)
    dst3d = jnp.concatenate([edge_index[1], dst_pad]).reshape(ERP, 1, LPR)
    zn = jnp.zeros((NP,), jnp.float32)
    zrows = jnp.zeros((NP, D), jnp.float32)

    cnt = _deg_call(src3d, dst3d, zn)
    z, ns, nd = _tc_first(n_feat, W1, cnt)
    p = _agg_call(z, src3d, dst3d, zrows)
    z = _tc_mid(p, nd, b1, W2, ns)
    p = _agg_call(z, src3d, dst3d, zrows)
    z = _tc_mid(p, nd, b2, W3, ns)
    p = _agg_call(z, src3d, dst3d, zrows)
    return _tc_final(p, nd, b3)


# speedup vs baseline: 1.0465x; 1.0017x over previous
import jax
import jax.numpy as jnp
from jax import lax
from jax.experimental import pallas as pl
from jax.experimental.pallas import tpu as pltpu
from jax.experimental.pallas import tpu_sc as plsc

N = 10000
E = 320000
D = 128
NC = 2
NS = 16
NW = NC * NS
LPR = 128
RW = 81
ERP = NW * RW
EP = ERP * LPR
NP = 10240
NT = NP // NS
RB = 632
RBL = N - RB * (NS - 1)

_mesh = plsc.VectorSubcoreMesh(
    core_axis_name="c", subcore_axis_name="s", num_cores=NC, num_subcores=NS
)


def _leaky(x):
    return jnp.where(x >= 0, x, 0.01 * x)


def _sc_degrees(src3d, dst3d, zn, cnt_out, is3, id3, ones_v, csrc, cdst, csem):
    c = lax.axis_index("c")
    s = lax.axis_index("s")
    wid = s * NC + c

    def fill(j, carry):
        ones_v[pl.ds(j * 16, 16)] = jnp.ones((16,), jnp.float32)
        return carry

    lax.fori_loop(0, LPR // 16, fill, 0)
    sl = pl.ds(s * NT, NT)
    pltpu.sync_copy(zn.at[sl], csrc.at[sl])
    pltpu.sync_copy(zn.at[sl], cdst.at[sl])
    plsc.subcore_barrier()

    base = wid * RW
    pltpu.sync_copy(src3d.at[pl.ds(base, RW)], is3)
    pltpu.sync_copy(dst3d.at[pl.ds(base, RW)], id3)

    def chunk(k, carry):
        i0 = 8 * k
        for j in range(8):
            pltpu.async_copy(ones_v, csrc.at[is3.at[i0 + j, 0]], csem,
                             add=True)
            pltpu.async_copy(ones_v, cdst.at[id3.at[i0 + j, 0]], csem,
                             add=True)
        for j in range(8):
            pltpu.make_async_copy(ones_v, csrc.at[is3.at[i0 + j, 0]],
                                  csem).wait()
            pltpu.make_async_copy(ones_v, cdst.at[id3.at[i0 + j, 0]],
                                  csem).wait()
        return carry

    lax.fori_loop(0, RW // 8, chunk, 0)
    for i in range(8 * (RW // 8), RW):
        pltpu.sync_copy(ones_v, csrc.at[is3.at[i, 0]], add=True)
        pltpu.sync_copy(ones_v, cdst.at[id3.at[i, 0]], add=True)
    plsc.subcore_barrier()
    pltpu.sync_copy(csrc.at[sl], cnt_out.at[0, c, sl])
    pltpu.sync_copy(cdst.at[sl], cnt_out.at[1, c, sl])


_deg_call = pl.kernel(
    _sc_degrees,
    out_type=jax.ShapeDtypeStruct((2, NC, NP), jnp.float32),
    mesh=_mesh,
    scratch_types=[
        pltpu.VMEM((RW, 1, LPR), jnp.int32),
        pltpu.VMEM((RW, 1, LPR), jnp.int32),
        pltpu.VMEM((LPR,), jnp.float32),
        pltpu.VMEM_SHARED((NP,), jnp.float32),
        pltpu.VMEM_SHARED((NP,), jnp.float32),
        pltpu.SemaphoreType.DMA,
    ],
)


def _sc_agg(z_hbm, src3d, dst3d, zrows, agg_out, ss3, dd3, b0, b1, b2,
            shared, g0, g1, g2, i0s, i1s, i2s, i3s, i4s, i5s, zs):
    c = lax.axis_index("c")
    s = lax.axis_index("s")
    wid = s * NC + c
    base = wid * RW

    @pl.when(s < NS - 1)
    def _():
        sl = pl.ds(s * RB, RB)
        pltpu.async_copy(zrows.at[sl], shared.at[sl], zs)

    @pl.when(s == NS - 1)
    def _():
        sl = pl.ds((NS - 1) * RB, RBL)
        pltpu.async_copy(zrows.at[sl], shared.at[sl], zs)

    bufs = (b0, b1, b2)
    gsems = (g0, g1, g2)
    isems = (i0s, i1s, i2s, i3s, i4s, i5s)

    def i_start(i, slot):
        pltpu.async_copy(src3d.at[base + i, 0], ss3.at[slot, 0], isems[slot])
        pltpu.async_copy(dst3d.at[base + i, 0], dd3.at[slot, 0], isems[slot])

    def i_wait(i, slot):
        pltpu.make_async_copy(src3d.at[base + i, 0], ss3.at[slot, 0],
                              isems[slot]).wait()
        pltpu.make_async_copy(dst3d.at[base + i, 0], dd3.at[slot, 0],
                              isems[slot]).wait()

    def g_start(slot, bj):
        pltpu.async_copy(z_hbm.at[ss3.at[slot, 0]], bufs[bj], gsems[bj])

    def g_wait(bj):
        pltpu.make_async_copy(z_hbm.at[ss3.at[0, 0]], bufs[bj],
                              gsems[bj]).wait()

    for j in range(6):
        i_start(j, j)
    for j in range(3):
        i_wait(j, j)
        g_start(j, j)

    @pl.when(s < NS - 1)
    def _():
        sl = pl.ds(s * RB, RB)
        pltpu.make_async_copy(zrows.at[sl], shared.at[sl], zs).wait()

    @pl.when(s == NS - 1)
    def _():
        sl = pl.ds((NS - 1) * RB, RBL)
        pltpu.make_async_copy(zrows.at[sl], shared.at[sl], zs).wait()

    plsc.subcore_barrier()

    def chunk(k, carry):
        ibase = 6 * k
        for u in range(6):
            i = ibase + u
            bj = u % 3
            g_wait(bj)
            pltpu.sync_copy(bufs[bj], shared.at[dd3.at[u, 0]], add=True)

            @pl.when(i + 3 < RW)
            def _(i=i, u=u, bj=bj):
                i_wait(i + 3, (u + 3) % 6)
                g_start((u + 3) % 6, bj)

            @pl.when(i + 6 < RW)
            def _(i=i, u=u):
                i_start(i + 6, u)

        return carry

    lax.fori_loop(0, RW // 6, chunk, 0)
    for u in range(RW - 6 * (RW // 6)):
        g_wait(u % 3)
        pltpu.sync_copy(bufs[u % 3], shared.at[dd3.at[u, 0]], add=True)

    plsc.subcore_barrier()

    @pl.when(s < NS - 1)
    def _():
        sl = pl.ds(s * RB, RB)
        pltpu.sync_copy(shared.at[sl], agg_out.at[c, sl])

    @pl.when(s == NS - 1)
    def _():
        sl = pl.ds((NS - 1) * RB, RBL)
        pltpu.sync_copy(shared.at[sl], agg_out.at[c, sl])


_agg_call = pl.kernel(
    _sc_agg,
    out_type=jax.ShapeDtypeStruct((NC, N, D), jnp.float32),
    mesh=_mesh,
    scratch_types=[
        pltpu.VMEM((6, 1, LPR), jnp.int32),
        pltpu.VMEM((6, 1, LPR), jnp.int32),
        pltpu.VMEM((LPR, D), jnp.float32),
        pltpu.VMEM((LPR, D), jnp.float32),
        pltpu.VMEM((LPR, D), jnp.float32),
        pltpu.VMEM_SHARED((N, D), jnp.float32),
    ] + [pltpu.SemaphoreType.DMA] * 10,
)


def _tc_first_body(x_ref, w_ref, cnt_ref, z_ref, ns_ref, nd_ref):
    cnt = cnt_ref[...]
    deg_o = cnt[0, 0, :N] + cnt[0, 1, :N]
    row = lax.broadcasted_iota(jnp.int32, (N,), 0)
    corr = jnp.where(row < (EP - E) % N, 2.0, 1.0)
    deg_i = cnt[1, 0, :N] + cnt[1, 1, :N] - corr
    ns = lax.rsqrt(jnp.maximum(deg_o, 1.0))[:, None]
    nd = lax.rsqrt(jnp.maximum(deg_i, 1.0))[:, None]
    u = jnp.dot(x_ref[...], w_ref[...], preferred_element_type=jnp.float32)
    z_ref[:N, :] = u * ns
    z_ref[N:, :] = jnp.zeros((NP - N, D), jnp.float32)
    ns_ref[...] = ns
    nd_ref[...] = nd


_tc_first = pl.pallas_call(
    _tc_first_body,
    out_shape=(
        jax.ShapeDtypeStruct((NP, D), jnp.float32),
        jax.ShapeDtypeStruct((N, 1), jnp.float32),
        jax.ShapeDtypeStruct((N, 1), jnp.float32),
    ),
)


def _tc_mid_body(p_ref, nd_ref, b_ref, w_ref, ns_ref, z_ref):
    agg = (p_ref[0] + p_ref[1]) * nd_ref[...]
    h = _leaky(agg + b_ref[...][None, :])
    z_ref[:N, :] = (
        jnp.dot(h, w_ref[...], preferred_element_type=jnp.float32)
        * ns_ref[...]
    )
    z_ref[N:, :] = jnp.zeros((NP - N, D), jnp.float32)


_tc_mid = pl.pallas_call(
    _tc_mid_body,
    out_shape=jax.ShapeDtypeStruct((NP, D), jnp.float32),
)


def _tc_final_body(p_ref, nd_ref, b_ref, o_ref):
    agg = (p_ref[0] + p_ref[1]) * nd_ref[...]
    o_ref[...] = _leaky(agg + b_ref[...][None, :])


_tc_final = pl.pallas_call(
    _tc_final_body,
    out_shape=jax.ShapeDtypeStruct((N, D), jnp.float32),
)


def kernel(n_feat, edge_index, W1, b1, W2, b2, W3, b3):
    npad = EP - E
    src_pad = N + (jnp.arange(npad, dtype=jnp.int32) % (NP - N))
    dst_pad = jnp.arange(npad, dtype=jnp.int32) % N
    src3d = jnp.concatenate([edge_index[0], src_pad]).reshape(ERP, 1, LPR)
    dst3d = jnp.concatenate([edge_index[1], dst_pad]).reshape(ERP, 1, LPR)
    zn = jnp.zeros((NP,), jnp.float32)
    zrows = jnp.zeros((NP, D), jnp.float32)

    cnt = _deg_call(src3d, dst3d, zn)
    z, ns, nd = _tc_first(n_feat, W1, cnt)
    p = _agg_call(z, src3d, dst3d, zrows)
    z = _tc_mid(p, nd, b1, W2, ns)
    p = _agg_call(z, src3d, dst3d, zrows)
    z = _tc_mid(p, nd, b2, W3, ns)
    p = _agg_call(z, src3d, dst3d, zrows)
    return _tc_final(p, nd, b3)


# speedup vs baseline: 1.0507x; 1.0040x over previous
import jax
import jax.numpy as jnp
from jax import lax
from jax.experimental import pallas as pl
from jax.experimental.pallas import tpu as pltpu
from jax.experimental.pallas import tpu_sc as plsc

N = 10000
E = 320000
D = 128
NC = 2
NS = 16
NW = NC * NS
LPR = 128
RW = 81
ERP = NW * RW
EP = ERP * LPR
NP = 10240
NT = NP // NS
RB = 632
RBL = N - RB * (NS - 1)

_mesh = plsc.VectorSubcoreMesh(
    core_axis_name="c", subcore_axis_name="s", num_cores=NC, num_subcores=NS
)


def _leaky(x):
    return jnp.where(x >= 0, x, 0.01 * x)


def _sc_degrees(src3d, dst3d, zn, cnt_out, is3, id3, ones_v, csrc, cdst, csem):
    c = lax.axis_index("c")
    s = lax.axis_index("s")
    wid = s * NC + c

    def fill(j, carry):
        ones_v[pl.ds(j * 16, 16)] = jnp.ones((16,), jnp.float32)
        return carry

    lax.fori_loop(0, LPR // 16, fill, 0)
    sl = pl.ds(s * NT, NT)
    pltpu.sync_copy(zn.at[sl], csrc.at[sl])
    pltpu.sync_copy(zn.at[sl], cdst.at[sl])
    plsc.subcore_barrier()

    base = wid * RW
    pltpu.sync_copy(src3d.at[pl.ds(base, RW)], is3)
    pltpu.sync_copy(dst3d.at[pl.ds(base, RW)], id3)

    def chunk(k, carry):
        i0 = 8 * k
        for j in range(8):
            pltpu.async_copy(ones_v, csrc.at[is3.at[i0 + j, 0]], csem,
                             add=True)
            pltpu.async_copy(ones_v, cdst.at[id3.at[i0 + j, 0]], csem,
                             add=True)
        for j in range(8):
            pltpu.make_async_copy(ones_v, csrc.at[is3.at[i0 + j, 0]],
                                  csem).wait()
            pltpu.make_async_copy(ones_v, cdst.at[id3.at[i0 + j, 0]],
                                  csem).wait()
        return carry

    lax.fori_loop(0, RW // 8, chunk, 0)
    for i in range(8 * (RW // 8), RW):
        pltpu.sync_copy(ones_v, csrc.at[is3.at[i, 0]], add=True)
        pltpu.sync_copy(ones_v, cdst.at[id3.at[i, 0]], add=True)
    plsc.subcore_barrier()
    pltpu.sync_copy(csrc.at[sl], cnt_out.at[0, c, sl])
    pltpu.sync_copy(cdst.at[sl], cnt_out.at[1, c, sl])


_deg_call = pl.kernel(
    _sc_degrees,
    out_type=jax.ShapeDtypeStruct((2, NC, NP), jnp.float32),
    mesh=_mesh,
    scratch_types=[
        pltpu.VMEM((RW, 1, LPR), jnp.int32),
        pltpu.VMEM((RW, 1, LPR), jnp.int32),
        pltpu.VMEM((LPR,), jnp.float32),
        pltpu.VMEM_SHARED((NP,), jnp.float32),
        pltpu.VMEM_SHARED((NP,), jnp.float32),
        pltpu.SemaphoreType.DMA,
    ],
)


def _sc_agg(z_hbm, src3d, dst3d, zrows, agg_out, ss3, dd3, b0, b1, b2,
            shared, g0, g1, g2, i0s, i1s, i2s, i3s, i4s, i5s, zs):
    c = lax.axis_index("c")
    s = lax.axis_index("s")
    wid = s * NC + c
    base = wid * RW

    @pl.when(s < NS - 1)
    def _():
        sl = pl.ds(s * RB, RB)
        pltpu.async_copy(zrows.at[sl], shared.at[sl], zs)

    @pl.when(s == NS - 1)
    def _():
        sl = pl.ds((NS - 1) * RB, RBL)
        pltpu.async_copy(zrows.at[sl], shared.at[sl], zs)

    bufs = (b0, b1, b2)
    gsems = (g0, g1, g2)
    isems = (i0s, i1s, i2s, i3s, i4s, i5s)

    def i_start(i, slot):
        pltpu.async_copy(src3d.at[base + i, 0], ss3.at[slot, 0], isems[slot])
        pltpu.async_copy(dst3d.at[base + i, 0], dd3.at[slot, 0], isems[slot])

    def i_wait(i, slot):
        pltpu.make_async_copy(src3d.at[base + i, 0], ss3.at[slot, 0],
                              isems[slot]).wait()
        pltpu.make_async_copy(dst3d.at[base + i, 0], dd3.at[slot, 0],
                              isems[slot]).wait()

    def g_start(slot, bj):
        pltpu.async_copy(z_hbm.at[ss3.at[slot, 0, pl.ds(0, 64)]],
                         bufs[bj].at[pl.ds(0, 64)], gsems[bj])
        pltpu.async_copy(z_hbm.at[ss3.at[slot, 0, pl.ds(64, 64)]],
                         bufs[bj].at[pl.ds(64, 64)], gsems[bj])

    def g_wait(bj):
        pltpu.make_async_copy(z_hbm.at[ss3.at[0, 0, pl.ds(0, 64)]],
                              bufs[bj].at[pl.ds(0, 64)], gsems[bj]).wait()
        pltpu.make_async_copy(z_hbm.at[ss3.at[0, 0, pl.ds(64, 64)]],
                              bufs[bj].at[pl.ds(64, 64)], gsems[bj]).wait()

    for j in range(6):
        i_start(j, j)
    for j in range(3):
        i_wait(j, j)
        g_start(j, j)

    @pl.when(s < NS - 1)
    def _():
        sl = pl.ds(s * RB, RB)
        pltpu.make_async_copy(zrows.at[sl], shared.at[sl], zs).wait()

    @pl.when(s == NS - 1)
    def _():
        sl = pl.ds((NS - 1) * RB, RBL)
        pltpu.make_async_copy(zrows.at[sl], shared.at[sl], zs).wait()

    plsc.subcore_barrier()

    def chunk(k, carry):
        ibase = 6 * k
        for u in range(6):
            i = ibase + u
            bj = u % 3
            g_wait(bj)
            pltpu.sync_copy(bufs[bj], shared.at[dd3.at[u, 0]], add=True)

            @pl.when(i + 3 < RW)
            def _(i=i, u=u, bj=bj):
                i_wait(i + 3, (u + 3) % 6)
                g_start((u + 3) % 6, bj)

            @pl.when(i + 6 < RW)
            def _(i=i, u=u):
                i_start(i + 6, u)

        return carry

    lax.fori_loop(0, RW // 6, chunk, 0)
    for u in range(RW - 6 * (RW // 6)):
        g_wait(u % 3)
        pltpu.sync_copy(bufs[u % 3], shared.at[dd3.at[u, 0]], add=True)

    plsc.subcore_barrier()

    @pl.when(s < NS - 1)
    def _():
        sl = pl.ds(s * RB, RB)
        pltpu.sync_copy(shared.at[sl], agg_out.at[c, sl])

    @pl.when(s == NS - 1)
    def _():
        sl = pl.ds((NS - 1) * RB, RBL)
        pltpu.sync_copy(shared.at[sl], agg_out.at[c, sl])


_agg_call = pl.kernel(
    _sc_agg,
    out_type=jax.ShapeDtypeStruct((NC, N, D), jnp.float32),
    mesh=_mesh,
    scratch_types=[
        pltpu.VMEM((6, 1, LPR), jnp.int32),
        pltpu.VMEM((6, 1, LPR), jnp.int32),
        pltpu.VMEM((LPR, D), jnp.float32),
        pltpu.VMEM((LPR, D), jnp.float32),
        pltpu.VMEM((LPR, D), jnp.float32),
        pltpu.VMEM_SHARED((N, D), jnp.float32),
    ] + [pltpu.SemaphoreType.DMA] * 10,
)


def _tc_first_body(x_ref, w_ref, cnt_ref, z_ref, ns_ref, nd_ref):
    cnt = cnt_ref[...]
    deg_o = cnt[0, 0, :N] + cnt[0, 1, :N]
    row = lax.broadcasted_iota(jnp.int32, (N,), 0)
    corr = jnp.where(row < (EP - E) % N, 2.0, 1.0)
    deg_i = cnt[1, 0, :N] + cnt[1, 1, :N] - corr
    ns = lax.rsqrt(jnp.maximum(deg_o, 1.0))[:, None]
    nd = lax.rsqrt(jnp.maximum(deg_i, 1.0))[:, None]
    u = jnp.dot(x_ref[...], w_ref[...], preferred_element_type=jnp.float32)
    z_ref[:N, :] = u * ns
    z_ref[N:, :] = jnp.zeros((NP - N, D), jnp.float32)
    ns_ref[...] = ns
    nd_ref[...] = nd


_tc_first = pl.pallas_call(
    _tc_first_body,
    out_shape=(
        jax.ShapeDtypeStruct((NP, D), jnp.float32),
        jax.ShapeDtypeStruct((N, 1), jnp.float32),
        jax.ShapeDtypeStruct((N, 1), jnp.float32),
    ),
)


def _tc_mid_body(p_ref, nd_ref, b_ref, w_ref, ns_ref, z_ref):
    agg = (p_ref[0] + p_ref[1]) * nd_ref[...]
    h = _leaky(agg + b_ref[...][None, :])
    z_ref[:N, :] = (
        jnp.dot(h, w_ref[...], preferred_element_type=jnp.float32)
        * ns_ref[...]
    )
    z_ref[N:, :] = jnp.zeros((NP - N, D), jnp.float32)


_tc_mid = pl.pallas_call(
    _tc_mid_body,
    out_shape=jax.ShapeDtypeStruct((NP, D), jnp.float32),
)


def _tc_final_body(p_ref, nd_ref, b_ref, o_ref):
    agg = (p_ref[0] + p_ref[1]) * nd_ref[...]
    o_ref[...] = _leaky(agg + b_ref[...][None, :])


_tc_final = pl.pallas_call(
    _tc_final_body,
    out_shape=jax.ShapeDtypeStruct((N, D), jnp.float32),
)


def kernel(n_feat, edge_index, W1, b1, W2, b2, W3, b3):
    npad = EP - E
    src_pad = N + (jnp.arange(npad, dtype=jnp.int32) % (NP - N))
    dst_pad = jnp.arange(npad, dtype=jnp.int32) % N
    src3d = jnp.concatenate([edge_index[0], src_pad]).reshape(ERP, 1, LPR)
    dst3d = jnp.concatenate([edge_index[1], dst_pad]).reshape(ERP, 1, LPR)
    zn = jnp.zeros((NP,), jnp.float32)
    zrows = jnp.zeros((NP, D), jnp.float32)

    cnt = _deg_call(src3d, dst3d, zn)
    z, ns, nd = _tc_first(n_feat, W1, cnt)
    p = _agg_call(z, src3d, dst3d, zrows)
    z = _tc_mid(p, nd, b1, W2, ns)
    p = _agg_call(z, src3d, dst3d, zrows)
    z = _tc_mid(p, nd, b2, W3, ns)
    p = _agg_call(z, src3d, dst3d, zrows)
    return _tc_final(p, nd, b3)
